# Initial kernel scaffold; baseline (speedup 1.0000x reference)
import jax
import jax.numpy as jnp
from jax.experimental import pallas as pl


def kernel(x, m, log_s, W, b):
    raise NotImplementedError("write your pallas kernel here")


# speedup vs baseline: 1.0188x; 1.0188x over previous
import functools
import math

import jax
import jax.numpy as jnp
import numpy as np
from jax.experimental import pallas as pl
from jax.experimental.pallas import tpu as pltpu

_TILE = 1024


def _body(x_ref, a_ref, off_ref, v_ref, nu_ref, qc_ref, s_ref, d_ref, o_ref):
    f32 = jnp.float32
    hi = jax.lax.Precision.HIGHEST
    xv = x_ref[...]

    logits = jax.lax.dot_general(
        xv, a_ref[...], (((1,), (1,)), ((), ())),
        preferred_element_type=f32, precision=hi) + off_ref[...]

    q = (jax.lax.dot_general(xv, v_ref[...], (((1,), (1,)), ((), ())),
                             preferred_element_type=f32, precision=hi)
         + jax.lax.dot_general(xv * xv, nu_ref[...], (((1,), (1,)), ((), ())),
                               preferred_element_type=f32, precision=hi)
         + qc_ref[...])

    gmax = jnp.max(logits, axis=-1, keepdims=True)
    e = jnp.exp(logits - gmax)
    ssum = jax.lax.dot_general(e, s_ref[...], (((1,), (0,)), ((), ())),
                               preferred_element_type=f32, precision=hi)
    lse = jnp.log(ssum) + gmax

    diag = jax.lax.dot_general(logits, d_ref[...], (((1,), (0,)), ((), ())),
                               preferred_element_type=f32, precision=hi)

    contrib = q + diag - lse
    cmax = jnp.max(contrib, axis=-1, keepdims=True)
    o_ref[...] = cmax + jnp.log(
        jnp.sum(jnp.exp(contrib - cmax), axis=-1, keepdims=True))


@functools.partial(jax.jit, static_argnames=())
def kernel(x, m, log_s, W, b):
    B, P = x.shape
    K = m.shape[0]
    f32 = jnp.float32

    inv_s = jnp.exp(-log_s)
    A = (inv_s[:, None, :] * W[None, :, :]).reshape(K * K, P)
    off = (b[None, :] - (m * inv_s) @ W.T).reshape(1, K * K)
    U = inv_s * inv_s
    V = m * U
    negU = -0.5 * U
    qc = (-0.5 * jnp.sum(m * m * U, axis=1)
          - 0.5 * P * math.log(2.0 * math.pi)
          - jnp.sum(log_s, axis=1)).reshape(1, K)

    lanes = np.arange(K * K)
    S = jnp.asarray((lanes[:, None] // K == np.arange(K)[None, :])
                    .astype(np.float32))
    D = jnp.asarray((lanes[:, None] == (K + 1) * np.arange(K)[None, :])
                    .astype(np.float32))

    tile = min(_TILE, B)
    grid = (B // tile,)
    rep = lambda shape: pl.BlockSpec(shape, lambda i: (0,) * len(shape))
    out = pl.pallas_call(
        _body,
        grid=grid,
        in_specs=[
            pl.BlockSpec((tile, P), lambda i: (i, 0)),
            rep((K * K, P)), rep((1, K * K)), rep((K, P)), rep((K, P)),
            rep((1, K)), rep((K * K, K)), rep((K * K, K)),
        ],
        out_specs=pl.BlockSpec((tile, 1), lambda i: (i, 0)),
        out_shape=jax.ShapeDtypeStruct((B, 1), f32),
        compiler_params=pltpu.CompilerParams(
            dimension_semantics=("arbitrary",)),
    )(x.astype(f32), A, off, V, negU, qc, S, D)
    return out.reshape(B)


# speedup vs baseline: 1.7300x; 1.6980x over previous
import functools
import math

import jax
import jax.numpy as jnp
import numpy as np
from jax.experimental import pallas as pl
from jax.experimental.pallas import tpu as pltpu

_TILE = 1024


def _body(x_ref, a_ref, off_ref, v_ref, nu_ref, ad_ref, qc_ref, s_ref, o_ref):
    f32 = jnp.float32
    hi = jax.lax.Precision.HIGHEST
    xv = x_ref[...]

    logits = jax.lax.dot_general(
        xv, a_ref[...], (((1,), (1,)), ((), ())),
        preferred_element_type=f32) + off_ref[...]

    q = (jax.lax.dot_general(xv, v_ref[...], (((1,), (1,)), ((), ())),
                             preferred_element_type=f32, precision=hi)
         + jax.lax.dot_general(xv * xv, nu_ref[...], (((1,), (1,)), ((), ())),
                               preferred_element_type=f32, precision=hi)
         + jax.lax.dot_general(xv, ad_ref[...], (((1,), (1,)), ((), ())),
                               preferred_element_type=f32, precision=hi)
         + qc_ref[...])

    gmax = jnp.max(logits, axis=-1, keepdims=True)
    e = jnp.exp(logits - gmax)
    ssum = jax.lax.dot_general(e, s_ref[...], (((1,), (0,)), ((), ())),
                               preferred_element_type=f32)
    lse = jnp.log(ssum) + gmax

    contrib = q - lse
    cmax = jnp.max(contrib, axis=-1, keepdims=True)
    o_ref[...] = cmax + jnp.log(
        jnp.sum(jnp.exp(contrib - cmax), axis=-1, keepdims=True))


@functools.partial(jax.jit, static_argnames=())
def kernel(x, m, log_s, W, b):
    B, P = x.shape
    K = m.shape[0]
    f32 = jnp.float32

    inv_s = jnp.exp(-log_s)
    A = (inv_s[:, None, :] * W[None, :, :]).reshape(K * K, P)
    offm = b[None, :] - (m * inv_s) @ W.T
    off = offm.reshape(1, K * K)
    U = inv_s * inv_s
    V = m * U
    negU = -0.5 * U
    Adiag = inv_s * W
    qc = (-0.5 * jnp.sum(m * m * U, axis=1)
          - 0.5 * P * math.log(2.0 * math.pi)
          - jnp.sum(log_s, axis=1)
          + jnp.diagonal(offm)).reshape(1, K)

    lanes = np.arange(K * K)
    S = jnp.asarray((lanes[:, None] // K == np.arange(K)[None, :])
                    .astype(np.float32))

    tile = min(_TILE, B)
    grid = (B // tile,)
    rep = lambda shape: pl.BlockSpec(shape, lambda i: (0,) * len(shape))
    out = pl.pallas_call(
        _body,
        grid=grid,
        in_specs=[
            pl.BlockSpec((tile, P), lambda i: (i, 0)),
            rep((K * K, P)), rep((1, K * K)), rep((K, P)), rep((K, P)),
            rep((K, P)), rep((1, K)), rep((K * K, K)),
        ],
        out_specs=pl.BlockSpec((tile, 1), lambda i: (i, 0)),
        out_shape=jax.ShapeDtypeStruct((B, 1), f32),
        compiler_params=pltpu.CompilerParams(
            dimension_semantics=("arbitrary",)),
    )(x.astype(f32), A, off, V, negU, Adiag, qc, S)
    return out.reshape(B)


# speedup vs baseline: 1.7888x; 1.0340x over previous
import functools
import math

import jax
import jax.numpy as jnp
import numpy as np
from jax.experimental import pallas as pl
from jax.experimental.pallas import tpu as pltpu

_TILE = 2048


def _body(x_ref, a_ref, off_ref, v_ref, nu_ref, ad_ref, qc_ref, s_ref, o_ref):
    f32 = jnp.float32
    hi = jax.lax.Precision.HIGHEST
    xv = x_ref[...]

    logits = jax.lax.dot_general(
        xv, a_ref[...], (((1,), (1,)), ((), ())),
        preferred_element_type=f32) + off_ref[...]

    q = (jax.lax.dot_general(xv, v_ref[...], (((1,), (1,)), ((), ())),
                             preferred_element_type=f32, precision=hi)
         + jax.lax.dot_general(xv * xv, nu_ref[...], (((1,), (1,)), ((), ())),
                               preferred_element_type=f32, precision=hi)
         + jax.lax.dot_general(xv, ad_ref[...], (((1,), (1,)), ((), ())),
                               preferred_element_type=f32, precision=hi)
         + qc_ref[...])

    gmax = jnp.max(logits, axis=-1, keepdims=True)
    e = jnp.exp(logits - gmax)
    ssum = jax.lax.dot_general(e, s_ref[...], (((1,), (0,)), ((), ())),
                               preferred_element_type=f32)
    lse = jnp.log(ssum) + gmax

    contrib = q - lse
    cmax = jnp.max(contrib, axis=-1, keepdims=True)
    o_ref[...] = cmax + jnp.log(
        jnp.sum(jnp.exp(contrib - cmax), axis=-1, keepdims=True))


@functools.partial(jax.jit, static_argnames=())
def kernel(x, m, log_s, W, b):
    B, P = x.shape
    K = m.shape[0]
    f32 = jnp.float32

    inv_s = jnp.exp(-log_s)
    A = (inv_s[:, None, :] * W[None, :, :]).reshape(K * K, P)
    offm = b[None, :] - (m * inv_s) @ W.T
    off = offm.reshape(1, K * K)
    U = inv_s * inv_s
    V = m * U
    negU = -0.5 * U
    Adiag = inv_s * W
    qc = (-0.5 * jnp.sum(m * m * U, axis=1)
          - 0.5 * P * math.log(2.0 * math.pi)
          - jnp.sum(log_s, axis=1)
          + jnp.diagonal(offm)).reshape(1, K)

    lanes = np.arange(K * K)
    S = jnp.asarray((lanes[:, None] // K == np.arange(K)[None, :])
                    .astype(np.float32))

    tile = min(_TILE, B)
    grid = (B // tile,)
    rep = lambda shape: pl.BlockSpec(shape, lambda i: (0,) * len(shape))
    out = pl.pallas_call(
        _body,
        grid=grid,
        in_specs=[
            pl.BlockSpec((tile, P), lambda i: (i, 0)),
            rep((K * K, P)), rep((1, K * K)), rep((K, P)), rep((K, P)),
            rep((K, P)), rep((1, K)), rep((K * K, K)),
        ],
        out_specs=pl.BlockSpec((tile, 1), lambda i: (i, 0)),
        out_shape=jax.ShapeDtypeStruct((B, 1), f32),
        compiler_params=pltpu.CompilerParams(
            dimension_semantics=("arbitrary",)),
    )(x.astype(f32), A, off, V, negU, Adiag, qc, S)
    return out.reshape(B)
